# baseline (speedup 1.0000x reference)
import functools

import jax
import jax.numpy as jnp
from jax import lax
from jax.experimental import pallas as pl
from jax.experimental.pallas import tpu as pltpu
from jax.experimental.pallas import tpu_sc as plsc

N = 10000
F = 128
E = 320000

NC = 2
NS = 16
NW = NC * NS
B = 100
EPW = E // NW
NB = EPW // B
N_PAD = 10240
RPT = N_PAD // NS

BM = 128
NBM = 80
E_PAD = NW * NBM * BM

_mesh = plsc.VectorSubcoreMesh(core_axis_name="c", subcore_axis_name="s")


@functools.partial(
    pl.kernel,
    out_type=(jax.ShapeDtypeStruct((NC, N_PAD), jnp.float32),
              jax.ShapeDtypeStruct((NC, N_PAD), jnp.float32)),
    mesh=_mesh,
    scratch_types=[
        pltpu.VMEM((NB, B), jnp.int32),
        pltpu.VMEM((NB, B), jnp.int32),
        pltpu.VMEM((128,), jnp.float32),
        pltpu.VMEM((RPT,), jnp.float32),
        pltpu.VMEM_SHARED((N_PAD,), jnp.float32),
        pltpu.VMEM_SHARED((N_PAD,), jnp.float32),
    ],
)
def _degree_kernel(src2d_hbm, dst2d_hbm, outs_hbm, outd_hbm,
                   src_v, dst_v, ones_v, zed_v, hsrc_sh, hdst_sh):
    c = lax.axis_index("c")
    s = lax.axis_index("s")
    wid = s * NC + c

    def _init(i, _):
        ones_v[pl.ds(i * 16, 16)] = jnp.full((16,), 1.0, jnp.float32)
        zed_v[pl.ds(i * 16, 16)] = jnp.zeros((16,), jnp.float32)
        return 0
    lax.fori_loop(0, 8, _init, 0)

    def _zed2(i, _):
        zed_v[pl.ds(128 + i * 16, 16)] = jnp.zeros((16,), jnp.float32)
        return 0
    lax.fori_loop(0, (RPT - 128) // 16, _zed2, 0)

    pltpu.sync_copy(zed_v, hsrc_sh.at[pl.ds(s * RPT, RPT)])
    pltpu.sync_copy(zed_v, hdst_sh.at[pl.ds(s * RPT, RPT)])
    plsc.subcore_barrier()

    pltpu.sync_copy(src2d_hbm.at[wid], src_v)
    pltpu.sync_copy(dst2d_hbm.at[wid], dst_v)

    def _step(j, _):
        pltpu.sync_copy(ones_v.at[pl.ds(0, B)], hsrc_sh.at[src_v.at[j]], add=True)
        pltpu.sync_copy(ones_v.at[pl.ds(0, B)], hdst_sh.at[dst_v.at[j]], add=True)
        return 0
    lax.fori_loop(0, NB, _step, 0)

    plsc.subcore_barrier()
    pltpu.sync_copy(hsrc_sh.at[pl.ds(s * RPT, RPT)], outs_hbm.at[c, pl.ds(s * RPT, RPT)])
    pltpu.sync_copy(hdst_sh.at[pl.ds(s * RPT, RPT)], outd_hbm.at[c, pl.ds(s * RPT, RPT)])


@functools.partial(
    pl.kernel,
    out_type=jax.ShapeDtypeStruct((NC, N_PAD, F), jnp.float32),
    mesh=_mesh,
    scratch_types=[
        pltpu.VMEM((2, BM), jnp.int32),
        pltpu.VMEM((2, BM), jnp.int32),
        pltpu.VMEM((BM, F), jnp.float32),
        pltpu.VMEM((BM, F), jnp.float32),
        pltpu.VMEM((16, F), jnp.float32),
        pltpu.VMEM_SHARED((N_PAD, F), jnp.float32),
        pltpu.SemaphoreType.DMA,
        pltpu.SemaphoreType.DMA,
        pltpu.SemaphoreType.DMA,
        pltpu.SemaphoreType.DMA,
    ],
)
def _msg_kernel(h_hbm, idx_hbm, out_hbm,
                idx0_v, idx1_v, rows0_v, rows1_v, z_v, acc_sh,
                semi0, semi1, semg0, semg1):
    c = lax.axis_index("c")
    s = lax.axis_index("s")
    wid = s * NC + c

    idxb = (idx0_v, idx1_v)
    rowsb = (rows0_v, rows1_v)
    semi = (semi0, semi1)
    semg = (semg0, semg1)

    def _zinit(i, _):
        for j in range(8):
            z_v[i, pl.ds(j * 16, 16)] = jnp.zeros((16,), jnp.float32)
        return 0
    lax.fori_loop(0, 16, _zinit, 0)

    def _zacc(i, _):
        pltpu.sync_copy(z_v, acc_sh.at[pl.ds(s * RPT + i * 16, 16)])
        return 0
    lax.fori_loop(0, RPT // 16, _zacc, 0)
    plsc.subcore_barrier()

    def _iload(j, p):
        pltpu.async_copy(idx_hbm.at[wid, j], idxb[p], semi[p])

    def _iwait(p):
        pltpu.make_async_copy(idx_hbm.at[wid, 0], idxb[p], semi[p]).wait()

    def _gather(p):
        pltpu.async_copy(h_hbm.at[idxb[p].at[0]], rowsb[p], semg[p])

    def _gwait(p):
        pltpu.make_async_copy(h_hbm.at[idxb[p].at[0]], rowsb[p], semg[p]).wait()

    def _scatter(p):
        pltpu.sync_copy(rowsb[p], acc_sh.at[idxb[p].at[1]], add=True)

    pltpu.sync_copy(idx_hbm.at[wid, 0], idx0_v)
    _gather(0)
    _iload(1, 1)

    def _stage(j, p):
        _iwait(1 - p)
        _gather(1 - p)
        _gwait(p)
        _scatter(p)
        _iload(jnp.minimum(j + 2, NBM - 1), p)

    def _pair(jj, _):
        j0 = 2 * jj
        _stage(j0, 0)
        _stage(j0 + 1, 1)
        return 0
    lax.fori_loop(0, NBM // 2, _pair, 0)

    _gwait(0)
    _iwait(1)

    plsc.subcore_barrier()
    pltpu.sync_copy(acc_sh.at[pl.ds(s * RPT, RPT)],
                    out_hbm.at[c, pl.ds(s * RPT, RPT)])


_R = 2000


def _pre_body(x_ref, d0_ref, d1_ref, w_ref, o_ref):
    deg = d0_ref[...] + d1_ref[...]
    norm = jnp.where(deg > 0, lax.rsqrt(jnp.maximum(deg, 1.0)), 0.0)
    o_ref[...] = jnp.dot(x_ref[...] * norm, w_ref[...],
                         preferred_element_type=jnp.float32)


def _tc_pre(x, dsrc0, dsrc1, W):
    return pl.pallas_call(
        _pre_body,
        grid=(N // _R,),
        in_specs=[
            pl.BlockSpec((_R, F), lambda i: (i, 0)),
            pl.BlockSpec((_R, 1), lambda i: (i, 0)),
            pl.BlockSpec((_R, 1), lambda i: (i, 0)),
            pl.BlockSpec((F, F), lambda i: (0, 0)),
        ],
        out_specs=pl.BlockSpec((_R, F), lambda i: (i, 0)),
        out_shape=jax.ShapeDtypeStruct((N, F), jnp.float32),
    )(x, dsrc0, dsrc1, W)


def _postpre_body(p0_ref, p1_ref, di0_ref, di1_ref, b_ref,
                  do0_ref, do1_ref, w_ref, h_ref, o_ref):
    degi = di0_ref[...] + di1_ref[...]
    normi = jnp.where(degi > 0, lax.rsqrt(jnp.maximum(degi, 1.0)), 0.0)
    h = jnp.maximum((p0_ref[0] + p1_ref[0]) * normi + b_ref[...], 0.0)
    h_ref[...] = h
    dego = do0_ref[...] + do1_ref[...]
    normo = jnp.where(dego > 0, lax.rsqrt(jnp.maximum(dego, 1.0)), 0.0)
    o_ref[...] = jnp.dot(h * normo, w_ref[...],
                         preferred_element_type=jnp.float32)


def _tc_postpre(partials, ddst0, ddst1, b, dsrc0, dsrc1, W):
    return pl.pallas_call(
        _postpre_body,
        grid=(N // _R,),
        in_specs=[
            pl.BlockSpec((1, _R, F), lambda i: (0, i, 0)),
            pl.BlockSpec((1, _R, F), lambda i: (1, i, 0)),
            pl.BlockSpec((_R, 1), lambda i: (i, 0)),
            pl.BlockSpec((_R, 1), lambda i: (i, 0)),
            pl.BlockSpec((1, F), lambda i: (0, 0)),
            pl.BlockSpec((_R, 1), lambda i: (i, 0)),
            pl.BlockSpec((_R, 1), lambda i: (i, 0)),
            pl.BlockSpec((F, F), lambda i: (0, 0)),
        ],
        out_specs=[
            pl.BlockSpec((_R, F), lambda i: (i, 0)),
            pl.BlockSpec((_R, F), lambda i: (i, 0)),
        ],
        out_shape=[
            jax.ShapeDtypeStruct((N, F), jnp.float32),
            jax.ShapeDtypeStruct((N, F), jnp.float32),
        ],
    )(partials, partials, ddst0, ddst1, b, dsrc0, dsrc1, W)


def _post_body(p0_ref, p1_ref, d0_ref, d1_ref, b_ref, o_ref, *, relu):
    deg = d0_ref[...] + d1_ref[...]
    norm = jnp.where(deg > 0, lax.rsqrt(jnp.maximum(deg, 1.0)), 0.0)
    h = (p0_ref[0] + p1_ref[0]) * norm + b_ref[...]
    if relu:
        h = jnp.maximum(h, 0.0)
    o_ref[...] = h


def _tc_post(partials, ddst0, ddst1, b, relu):
    return pl.pallas_call(
        functools.partial(_post_body, relu=relu),
        grid=(N // _R,),
        in_specs=[
            pl.BlockSpec((1, _R, F), lambda i: (0, i, 0)),
            pl.BlockSpec((1, _R, F), lambda i: (1, i, 0)),
            pl.BlockSpec((_R, 1), lambda i: (i, 0)),
            pl.BlockSpec((_R, 1), lambda i: (i, 0)),
            pl.BlockSpec((1, F), lambda i: (0, 0)),
        ],
        out_specs=pl.BlockSpec((_R, F), lambda i: (i, 0)),
        out_shape=jax.ShapeDtypeStruct((N, F), jnp.float32),
    )(partials, partials, ddst0, ddst1, b)


def kernel(inputs, edge_index, embedding_layer, W1, b1, W2, b2, W3, b3):
    src2d = edge_index[0].reshape(NW, NB, B)
    dst2d = edge_index[1].reshape(NW, NB, B)
    src_pad = jnp.concatenate(
        [edge_index[0], jnp.zeros((E_PAD - E,), jnp.int32)])
    dst_pad = jnp.concatenate(
        [edge_index[1],
         N + (jnp.arange(E_PAD - E, dtype=jnp.int32) % (N_PAD - N))])
    idx4 = jnp.stack([src_pad, dst_pad]).reshape(2, NW, NBM, BM).transpose(1, 2, 0, 3)

    dsrc_p, ddst_p = _degree_kernel(src2d, dst2d)
    dsrc0 = dsrc_p[0, :N].reshape(N, 1)
    dsrc1 = dsrc_p[1, :N].reshape(N, 1)
    ddst0 = ddst_p[0, :N].reshape(N, 1)
    ddst1 = ddst_p[1, :N].reshape(N, 1)

    b1r = b1.reshape(1, F)
    b2r = b2.reshape(1, F)
    b3r = b3.reshape(1, F)

    pre1 = _tc_pre(inputs, dsrc0, dsrc1, W1)
    p1_ = _msg_kernel(pre1, idx4)
    h1, pre2 = _tc_postpre(p1_, ddst0, ddst1, b1r, dsrc0, dsrc1, W2)

    p2_ = _msg_kernel(pre2, idx4)
    h2, pre3 = _tc_postpre(p2_, ddst0, ddst1, b2r, dsrc0, dsrc1, W3)

    p3_ = _msg_kernel(pre3, idx4)
    h3 = _tc_post(p3_, ddst0, ddst1, b3r, relu=False)

    emb = jnp.where(embedding_layer == 1, h1,
                    jnp.where(embedding_layer == 2, h2, h3))
    return (h3, emb, inputs)


# speedup vs baseline: 2.9545x; 2.9545x over previous
import functools

import jax
import jax.numpy as jnp
from jax import lax
from jax.experimental import pallas as pl
from jax.experimental.pallas import tpu as pltpu
from jax.experimental.pallas import tpu_sc as plsc

N = 10000
F = 128
E = 320000

NC = 2
NS = 16
NW = NC * NS
B = 100
EPW = E // NW
NB = EPW // B
N_PAD = 10240
RPT = N_PAD // NS

BM = 100
NBM = 100
E_PAD = NW * NBM * BM

_mesh = plsc.VectorSubcoreMesh(core_axis_name="c", subcore_axis_name="s")


@functools.partial(
    pl.kernel,
    out_type=(jax.ShapeDtypeStruct((NC, N_PAD), jnp.float32),
              jax.ShapeDtypeStruct((NC, N_PAD), jnp.float32)),
    mesh=_mesh,
    scratch_types=[
        pltpu.VMEM((NB, B), jnp.int32),
        pltpu.VMEM((NB, B), jnp.int32),
        pltpu.VMEM((128,), jnp.float32),
        pltpu.VMEM((RPT,), jnp.float32),
        pltpu.VMEM_SHARED((N_PAD,), jnp.float32),
        pltpu.VMEM_SHARED((N_PAD,), jnp.float32),
    ],
)
def _degree_kernel(src2d_hbm, dst2d_hbm, outs_hbm, outd_hbm,
                   src_v, dst_v, ones_v, zed_v, hsrc_sh, hdst_sh):
    c = lax.axis_index("c")
    s = lax.axis_index("s")
    wid = s * NC + c

    def _init(i, _):
        ones_v[pl.ds(i * 16, 16)] = jnp.full((16,), 1.0, jnp.float32)
        zed_v[pl.ds(i * 16, 16)] = jnp.zeros((16,), jnp.float32)
        return 0
    lax.fori_loop(0, 8, _init, 0)

    def _zed2(i, _):
        zed_v[pl.ds(128 + i * 16, 16)] = jnp.zeros((16,), jnp.float32)
        return 0
    lax.fori_loop(0, (RPT - 128) // 16, _zed2, 0)

    pltpu.sync_copy(zed_v, hsrc_sh.at[pl.ds(s * RPT, RPT)])
    pltpu.sync_copy(zed_v, hdst_sh.at[pl.ds(s * RPT, RPT)])
    plsc.subcore_barrier()

    pltpu.sync_copy(src2d_hbm.at[wid], src_v)
    pltpu.sync_copy(dst2d_hbm.at[wid], dst_v)

    def _step(j, _):
        pltpu.sync_copy(ones_v.at[pl.ds(0, B)], hsrc_sh.at[src_v.at[j]], add=True)
        pltpu.sync_copy(ones_v.at[pl.ds(0, B)], hdst_sh.at[dst_v.at[j]], add=True)
        return 0
    lax.fori_loop(0, NB, _step, 0)

    plsc.subcore_barrier()
    pltpu.sync_copy(hsrc_sh.at[pl.ds(s * RPT, RPT)], outs_hbm.at[c, pl.ds(s * RPT, RPT)])
    pltpu.sync_copy(hdst_sh.at[pl.ds(s * RPT, RPT)], outd_hbm.at[c, pl.ds(s * RPT, RPT)])


@functools.partial(
    pl.kernel,
    out_type=jax.ShapeDtypeStruct((NC, N_PAD, F), jnp.float32),
    mesh=_mesh,
    scratch_types=[
        pltpu.VMEM((2, BM), jnp.int32),
        pltpu.VMEM((2, BM), jnp.int32),
        pltpu.VMEM((BM, F), jnp.float32),
        pltpu.VMEM((BM, F), jnp.float32),
        pltpu.VMEM((16, F), jnp.float32),
        pltpu.VMEM_SHARED((N_PAD, F), jnp.float32),
        pltpu.SemaphoreType.DMA,
        pltpu.SemaphoreType.DMA,
        pltpu.SemaphoreType.DMA,
        pltpu.SemaphoreType.DMA,
    ],
)
def _msg_kernel(h_hbm, idx_hbm, out_hbm,
                idx0_v, idx1_v, rows0_v, rows1_v, z_v, acc_sh,
                semi0, semi1, semg0, semg1):
    c = lax.axis_index("c")
    s = lax.axis_index("s")
    wid = s * NC + c

    idxb = (idx0_v, idx1_v)
    rowsb = (rows0_v, rows1_v)
    semi = (semi0, semi1)
    semg = (semg0, semg1)

    def _zinit(i, _):
        for j in range(8):
            z_v[i, pl.ds(j * 16, 16)] = jnp.zeros((16,), jnp.float32)
        return 0
    lax.fori_loop(0, 16, _zinit, 0)

    def _zacc(i, _):
        pltpu.sync_copy(z_v, acc_sh.at[pl.ds(s * RPT + i * 16, 16)])
        return 0
    lax.fori_loop(0, RPT // 16, _zacc, 0)
    plsc.subcore_barrier()

    def _iload(j, p):
        pltpu.async_copy(idx_hbm.at[wid, j], idxb[p], semi[p])

    def _iwait(p):
        pltpu.make_async_copy(idx_hbm.at[wid, 0], idxb[p], semi[p]).wait()

    def _gather(p):
        pltpu.async_copy(h_hbm.at[idxb[p].at[0]], rowsb[p], semg[p])

    def _gwait(p):
        pltpu.make_async_copy(h_hbm.at[idxb[p].at[0]], rowsb[p], semg[p]).wait()

    def _scatter(p):
        pltpu.sync_copy(rowsb[p], acc_sh.at[idxb[p].at[1]], add=True)

    pltpu.sync_copy(idx_hbm.at[wid, 0], idx0_v)
    _gather(0)
    _iload(1, 1)

    def _stage(j, p):
        _iwait(1 - p)
        _gather(1 - p)
        _gwait(p)
        _scatter(p)
        _iload(jnp.minimum(j + 2, NBM - 1), p)

    def _pair(jj, _):
        j0 = 2 * jj
        _stage(j0, 0)
        _stage(j0 + 1, 1)
        return 0
    lax.fori_loop(0, NBM // 2, _pair, 0)

    _gwait(0)
    _iwait(1)

    plsc.subcore_barrier()
    pltpu.sync_copy(acc_sh.at[pl.ds(s * RPT, RPT)],
                    out_hbm.at[c, pl.ds(s * RPT, RPT)])


_R = 2000


def _pre_body(x_ref, d0_ref, d1_ref, w_ref, o_ref):
    deg = d0_ref[...] + d1_ref[...]
    norm = jnp.where(deg > 0, lax.rsqrt(jnp.maximum(deg, 1.0)), 0.0)
    o_ref[...] = jnp.dot(x_ref[...] * norm, w_ref[...],
                         preferred_element_type=jnp.float32)


def _tc_pre(x, dsrc0, dsrc1, W):
    return pl.pallas_call(
        _pre_body,
        grid=(N // _R,),
        in_specs=[
            pl.BlockSpec((_R, F), lambda i: (i, 0)),
            pl.BlockSpec((_R, 1), lambda i: (i, 0)),
            pl.BlockSpec((_R, 1), lambda i: (i, 0)),
            pl.BlockSpec((F, F), lambda i: (0, 0)),
        ],
        out_specs=pl.BlockSpec((_R, F), lambda i: (i, 0)),
        out_shape=jax.ShapeDtypeStruct((N, F), jnp.float32),
    )(x, dsrc0, dsrc1, W)


def _postpre_body(p0_ref, p1_ref, di0_ref, di1_ref, b_ref,
                  do0_ref, do1_ref, w_ref, h_ref, o_ref):
    degi = di0_ref[...] + di1_ref[...]
    normi = jnp.where(degi > 0, lax.rsqrt(jnp.maximum(degi, 1.0)), 0.0)
    h = jnp.maximum((p0_ref[0] + p1_ref[0]) * normi + b_ref[...], 0.0)
    h_ref[...] = h
    dego = do0_ref[...] + do1_ref[...]
    normo = jnp.where(dego > 0, lax.rsqrt(jnp.maximum(dego, 1.0)), 0.0)
    o_ref[...] = jnp.dot(h * normo, w_ref[...],
                         preferred_element_type=jnp.float32)


def _tc_postpre(partials, ddst0, ddst1, b, dsrc0, dsrc1, W):
    return pl.pallas_call(
        _postpre_body,
        grid=(N // _R,),
        in_specs=[
            pl.BlockSpec((1, _R, F), lambda i: (0, i, 0)),
            pl.BlockSpec((1, _R, F), lambda i: (1, i, 0)),
            pl.BlockSpec((_R, 1), lambda i: (i, 0)),
            pl.BlockSpec((_R, 1), lambda i: (i, 0)),
            pl.BlockSpec((1, F), lambda i: (0, 0)),
            pl.BlockSpec((_R, 1), lambda i: (i, 0)),
            pl.BlockSpec((_R, 1), lambda i: (i, 0)),
            pl.BlockSpec((F, F), lambda i: (0, 0)),
        ],
        out_specs=[
            pl.BlockSpec((_R, F), lambda i: (i, 0)),
            pl.BlockSpec((_R, F), lambda i: (i, 0)),
        ],
        out_shape=[
            jax.ShapeDtypeStruct((N, F), jnp.float32),
            jax.ShapeDtypeStruct((N, F), jnp.float32),
        ],
    )(partials, partials, ddst0, ddst1, b, dsrc0, dsrc1, W)


def _post_body(p0_ref, p1_ref, d0_ref, d1_ref, b_ref, o_ref, *, relu):
    deg = d0_ref[...] + d1_ref[...]
    norm = jnp.where(deg > 0, lax.rsqrt(jnp.maximum(deg, 1.0)), 0.0)
    h = (p0_ref[0] + p1_ref[0]) * norm + b_ref[...]
    if relu:
        h = jnp.maximum(h, 0.0)
    o_ref[...] = h


def _tc_post(partials, ddst0, ddst1, b, relu):
    return pl.pallas_call(
        functools.partial(_post_body, relu=relu),
        grid=(N // _R,),
        in_specs=[
            pl.BlockSpec((1, _R, F), lambda i: (0, i, 0)),
            pl.BlockSpec((1, _R, F), lambda i: (1, i, 0)),
            pl.BlockSpec((_R, 1), lambda i: (i, 0)),
            pl.BlockSpec((_R, 1), lambda i: (i, 0)),
            pl.BlockSpec((1, F), lambda i: (0, 0)),
        ],
        out_specs=pl.BlockSpec((_R, F), lambda i: (i, 0)),
        out_shape=jax.ShapeDtypeStruct((N, F), jnp.float32),
    )(partials, partials, ddst0, ddst1, b)


def kernel(inputs, edge_index, embedding_layer, W1, b1, W2, b2, W3, b3):
    src2d = edge_index[0].reshape(NW, NB, B)
    dst2d = edge_index[1].reshape(NW, NB, B)
    idx4 = edge_index.reshape(2, NW, NBM, BM).transpose(1, 2, 0, 3)

    dsrc_p, ddst_p = _degree_kernel(src2d, dst2d)
    dsrc0 = dsrc_p[0, :N].reshape(N, 1)
    dsrc1 = dsrc_p[1, :N].reshape(N, 1)
    ddst0 = ddst_p[0, :N].reshape(N, 1)
    ddst1 = ddst_p[1, :N].reshape(N, 1)

    b1r = b1.reshape(1, F)
    b2r = b2.reshape(1, F)
    b3r = b3.reshape(1, F)

    pre1 = _tc_pre(inputs, dsrc0, dsrc1, W1)
    p1_ = _msg_kernel(pre1, idx4)
    h1, pre2 = _tc_postpre(p1_, ddst0, ddst1, b1r, dsrc0, dsrc1, W2)

    p2_ = _msg_kernel(pre2, idx4)
    h2, pre3 = _tc_postpre(p2_, ddst0, ddst1, b2r, dsrc0, dsrc1, W3)

    p3_ = _msg_kernel(pre3, idx4)
    h3 = _tc_post(p3_, ddst0, ddst1, b3r, relu=False)

    emb = jnp.where(embedding_layer == 1, h1,
                    jnp.where(embedding_layer == 2, h2, h3))
    return (h3, emb, inputs)


# speedup vs baseline: 3.3517x; 1.1345x over previous
import functools

import jax
import jax.numpy as jnp
from jax import lax
from jax.experimental import pallas as pl
from jax.experimental.pallas import tpu as pltpu
from jax.experimental.pallas import tpu_sc as plsc

N = 10000
F = 128
E = 320000

NC = 2
NS = 16
NW = NC * NS
B = 100
EPW = E // NW
NB = EPW // B
N_PAD = 10240
RPT = N_PAD // NS

BM = 100
NBM = 100
E_PAD = NW * NBM * BM

_mesh = plsc.VectorSubcoreMesh(core_axis_name="c", subcore_axis_name="s")


@functools.partial(
    pl.kernel,
    out_type=(jax.ShapeDtypeStruct((NC, N_PAD), jnp.float32),
              jax.ShapeDtypeStruct((NC, N_PAD), jnp.float32)),
    mesh=_mesh,
    scratch_types=[
        pltpu.VMEM((NB, B), jnp.int32),
        pltpu.VMEM((NB, B), jnp.int32),
        pltpu.VMEM((128,), jnp.float32),
        pltpu.VMEM((RPT,), jnp.float32),
        pltpu.VMEM_SHARED((N_PAD,), jnp.float32),
        pltpu.VMEM_SHARED((N_PAD,), jnp.float32),
    ],
)
def _degree_kernel(src2d_hbm, dst2d_hbm, outs_hbm, outd_hbm,
                   src_v, dst_v, ones_v, zed_v, hsrc_sh, hdst_sh):
    c = lax.axis_index("c")
    s = lax.axis_index("s")
    wid = s * NC + c

    def _init(i, _):
        ones_v[pl.ds(i * 16, 16)] = jnp.full((16,), 1.0, jnp.float32)
        zed_v[pl.ds(i * 16, 16)] = jnp.zeros((16,), jnp.float32)
        return 0
    lax.fori_loop(0, 8, _init, 0)

    def _zed2(i, _):
        zed_v[pl.ds(128 + i * 16, 16)] = jnp.zeros((16,), jnp.float32)
        return 0
    lax.fori_loop(0, (RPT - 128) // 16, _zed2, 0)

    pltpu.sync_copy(zed_v, hsrc_sh.at[pl.ds(s * RPT, RPT)])
    pltpu.sync_copy(zed_v, hdst_sh.at[pl.ds(s * RPT, RPT)])
    plsc.subcore_barrier()

    pltpu.sync_copy(src2d_hbm.at[wid], src_v)
    pltpu.sync_copy(dst2d_hbm.at[wid], dst_v)

    def _step(j, _):
        pltpu.sync_copy(ones_v.at[pl.ds(0, B)], hsrc_sh.at[src_v.at[j]], add=True)
        pltpu.sync_copy(ones_v.at[pl.ds(0, B)], hdst_sh.at[dst_v.at[j]], add=True)
        return 0
    lax.fori_loop(0, NB, _step, 0)

    plsc.subcore_barrier()
    pltpu.sync_copy(hsrc_sh.at[pl.ds(s * RPT, RPT)], outs_hbm.at[c, pl.ds(s * RPT, RPT)])
    pltpu.sync_copy(hdst_sh.at[pl.ds(s * RPT, RPT)], outd_hbm.at[c, pl.ds(s * RPT, RPT)])


@functools.partial(
    pl.kernel,
    out_type=jax.ShapeDtypeStruct((NC, N_PAD, F), jnp.float32),
    mesh=_mesh,
    scratch_types=[
        pltpu.VMEM((2, BM), jnp.int32),
        pltpu.VMEM((2, BM), jnp.int32),
        pltpu.VMEM((2, BM), jnp.int32),
        pltpu.VMEM((2, BM), jnp.int32),
        pltpu.VMEM((BM, F), jnp.float32),
        pltpu.VMEM((BM, F), jnp.float32),
        pltpu.VMEM((16, F), jnp.float32),
        pltpu.VMEM_SHARED((N_PAD, F), jnp.float32),
        pltpu.SemaphoreType.DMA,
        pltpu.SemaphoreType.DMA,
        pltpu.SemaphoreType.DMA,
        pltpu.SemaphoreType.DMA,
        pltpu.SemaphoreType.DMA,
        pltpu.SemaphoreType.DMA,
        pltpu.SemaphoreType.DMA,
        pltpu.SemaphoreType.DMA,
    ],
)
def _msg_kernel(h_hbm, idx_hbm, out_hbm,
                idx0_v, idx1_v, idx2_v, idx3_v, rows0_v, rows1_v, z_v, acc_sh,
                semi0, semi1, semi2, semi3, semg0, semg1, sems0, sems1):
    c = lax.axis_index("c")
    s = lax.axis_index("s")
    wid = s * NC + c

    idxb = (idx0_v, idx1_v, idx2_v, idx3_v)
    rowsb = (rows0_v, rows1_v)
    semi = (semi0, semi1, semi2, semi3)
    semg = (semg0, semg1)
    sems = (sems0, sems1)

    def _zinit(i, _):
        for j in range(8):
            z_v[i, pl.ds(j * 16, 16)] = jnp.zeros((16,), jnp.float32)
        return 0
    lax.fori_loop(0, 16, _zinit, 0)

    def _zacc(i, _):
        pltpu.sync_copy(z_v, acc_sh.at[pl.ds(s * RPT + i * 16, 16)])
        return 0
    lax.fori_loop(0, RPT // 16, _zacc, 0)
    plsc.subcore_barrier()

    def _iload(j, q):
        pltpu.async_copy(idx_hbm.at[wid, j], idxb[q], semi[q])

    def _iwait(q):
        pltpu.make_async_copy(idx_hbm.at[wid, 0], idxb[q], semi[q]).wait()

    def _gather(q, p):
        pltpu.async_copy(h_hbm.at[idxb[q].at[0]], rowsb[p], semg[p])

    def _gwait(q, p):
        pltpu.make_async_copy(h_hbm.at[idxb[q].at[0]], rowsb[p], semg[p]).wait()

    def _scatter(q, p):
        pltpu.async_copy(rowsb[p], acc_sh.at[idxb[q].at[1]], sems[p], add=True)

    def _swait(q, p):
        pltpu.make_async_copy(rowsb[p], acc_sh.at[idxb[q].at[1]], sems[p]).wait()

    def _stage(j, u, first=False, last=False):
        p = u % 2
        q = u % 4
        if not last:
            _iwait((u + 1) % 4)
        if not first:
            _swait((u + 3) % 4, 1 - p)
        if not last:
            _gather((u + 1) % 4, 1 - p)
        _gwait(q, p)
        _scatter(q, p)
        if not last:
            _iload(jnp.minimum(j + 2, NBM - 1), (u + 2) % 4)

    pltpu.sync_copy(idx_hbm.at[wid, 0], idx0_v)
    _gather(0, 0)
    _iload(1, 1)

    _stage(0, 0, first=True)
    _stage(1, 1)
    _stage(2, 2)
    _stage(3, 3)

    def _quad(jjj, _):
        j0 = 4 * jjj
        _stage(j0, 0)
        _stage(j0 + 1, 1)
        _stage(j0 + 2, 2)
        _stage(j0 + 3, 3)
        return 0
    lax.fori_loop(1, NBM // 4 - 1, _quad, 0)

    _stage(NBM - 4, 0)
    _stage(NBM - 3, 1)
    _stage(NBM - 2, 2)
    _iwait((NBM - 1 + 1) % 4)
    _stage(NBM - 1, 3, last=True)
    _swait(3, 1)

    plsc.subcore_barrier()
    pltpu.sync_copy(acc_sh.at[pl.ds(s * RPT, RPT)],
                    out_hbm.at[c, pl.ds(s * RPT, RPT)])


_R = 2000


def _pre_body(x_ref, d0_ref, d1_ref, w_ref, o_ref):
    deg = d0_ref[...] + d1_ref[...]
    norm = jnp.where(deg > 0, lax.rsqrt(jnp.maximum(deg, 1.0)), 0.0)
    o_ref[...] = jnp.dot(x_ref[...] * norm, w_ref[...],
                         preferred_element_type=jnp.float32)


def _tc_pre(x, dsrc0, dsrc1, W):
    return pl.pallas_call(
        _pre_body,
        grid=(N // _R,),
        in_specs=[
            pl.BlockSpec((_R, F), lambda i: (i, 0)),
            pl.BlockSpec((_R, 1), lambda i: (i, 0)),
            pl.BlockSpec((_R, 1), lambda i: (i, 0)),
            pl.BlockSpec((F, F), lambda i: (0, 0)),
        ],
        out_specs=pl.BlockSpec((_R, F), lambda i: (i, 0)),
        out_shape=jax.ShapeDtypeStruct((N, F), jnp.float32),
    )(x, dsrc0, dsrc1, W)


def _postpre_body(p0_ref, p1_ref, di0_ref, di1_ref, b_ref,
                  do0_ref, do1_ref, w_ref, h_ref, o_ref):
    degi = di0_ref[...] + di1_ref[...]
    normi = jnp.where(degi > 0, lax.rsqrt(jnp.maximum(degi, 1.0)), 0.0)
    p = p0_ref[0].astype(jnp.float32) + p1_ref[0].astype(jnp.float32)
    h = jnp.maximum(p * normi + b_ref[...], 0.0)
    h_ref[...] = h
    dego = do0_ref[...] + do1_ref[...]
    normo = jnp.where(dego > 0, lax.rsqrt(jnp.maximum(dego, 1.0)), 0.0)
    o_ref[...] = jnp.dot(h * normo, w_ref[...],
                         preferred_element_type=jnp.float32)


def _tc_postpre(partials, ddst0, ddst1, b, dsrc0, dsrc1, W):
    return pl.pallas_call(
        _postpre_body,
        grid=(N // _R,),
        in_specs=[
            pl.BlockSpec((1, _R, F), lambda i: (0, i, 0)),
            pl.BlockSpec((1, _R, F), lambda i: (1, i, 0)),
            pl.BlockSpec((_R, 1), lambda i: (i, 0)),
            pl.BlockSpec((_R, 1), lambda i: (i, 0)),
            pl.BlockSpec((1, F), lambda i: (0, 0)),
            pl.BlockSpec((_R, 1), lambda i: (i, 0)),
            pl.BlockSpec((_R, 1), lambda i: (i, 0)),
            pl.BlockSpec((F, F), lambda i: (0, 0)),
        ],
        out_specs=[
            pl.BlockSpec((_R, F), lambda i: (i, 0)),
            pl.BlockSpec((_R, F), lambda i: (i, 0)),
        ],
        out_shape=[
            jax.ShapeDtypeStruct((N, F), jnp.float32),
            jax.ShapeDtypeStruct((N, F), jnp.float32),
        ],
    )(partials, partials, ddst0, ddst1, b, dsrc0, dsrc1, W)


def _post_body(p0_ref, p1_ref, d0_ref, d1_ref, b_ref, o_ref, *, relu):
    deg = d0_ref[...] + d1_ref[...]
    norm = jnp.where(deg > 0, lax.rsqrt(jnp.maximum(deg, 1.0)), 0.0)
    p = p0_ref[0].astype(jnp.float32) + p1_ref[0].astype(jnp.float32)
    h = p * norm + b_ref[...]
    if relu:
        h = jnp.maximum(h, 0.0)
    o_ref[...] = h


def _tc_post(partials, ddst0, ddst1, b, relu):
    return pl.pallas_call(
        functools.partial(_post_body, relu=relu),
        grid=(N // _R,),
        in_specs=[
            pl.BlockSpec((1, _R, F), lambda i: (0, i, 0)),
            pl.BlockSpec((1, _R, F), lambda i: (1, i, 0)),
            pl.BlockSpec((_R, 1), lambda i: (i, 0)),
            pl.BlockSpec((_R, 1), lambda i: (i, 0)),
            pl.BlockSpec((1, F), lambda i: (0, 0)),
        ],
        out_specs=pl.BlockSpec((_R, F), lambda i: (i, 0)),
        out_shape=jax.ShapeDtypeStruct((N, F), jnp.float32),
    )(partials, partials, ddst0, ddst1, b)


def kernel(inputs, edge_index, embedding_layer, W1, b1, W2, b2, W3, b3):
    src2d = edge_index[0].reshape(NW, NB, B)
    dst2d = edge_index[1].reshape(NW, NB, B)
    idx4 = edge_index.reshape(2, NW, NBM, BM).transpose(1, 2, 0, 3)

    dsrc_p, ddst_p = _degree_kernel(src2d, dst2d)
    dsrc0 = dsrc_p[0, :N].reshape(N, 1)
    dsrc1 = dsrc_p[1, :N].reshape(N, 1)
    ddst0 = ddst_p[0, :N].reshape(N, 1)
    ddst1 = ddst_p[1, :N].reshape(N, 1)

    b1r = b1.reshape(1, F)
    b2r = b2.reshape(1, F)
    b3r = b3.reshape(1, F)

    pre1 = _tc_pre(inputs, dsrc0, dsrc1, W1)
    p1_ = _msg_kernel(pre1, idx4)
    h1, pre2 = _tc_postpre(p1_, ddst0, ddst1, b1r, dsrc0, dsrc1, W2)

    p2_ = _msg_kernel(pre2, idx4)
    h2, pre3 = _tc_postpre(p2_, ddst0, ddst1, b2r, dsrc0, dsrc1, W3)

    p3_ = _msg_kernel(pre3, idx4)
    h3 = _tc_post(p3_, ddst0, ddst1, b3r, relu=False)

    emb = jnp.where(embedding_layer == 1, h1,
                    jnp.where(embedding_layer == 2, h2, h3))
    return (h3, emb, inputs)


# speedup vs baseline: 3.4761x; 1.0371x over previous
import functools

import jax
import jax.numpy as jnp
from jax import lax
from jax.experimental import pallas as pl
from jax.experimental.pallas import tpu as pltpu
from jax.experimental.pallas import tpu_sc as plsc

N = 10000
F = 128
E = 320000

NC = 2
NS = 16
NW = NC * NS
B = 100
EPW = E // NW
NB = EPW // B
N_PAD = 10240
RPT = N_PAD // NS

BM = 100
NBM = 100
E_PAD = NW * NBM * BM

_mesh = plsc.VectorSubcoreMesh(core_axis_name="c", subcore_axis_name="s")


@functools.partial(
    pl.kernel,
    out_type=(jax.ShapeDtypeStruct((NC, N_PAD), jnp.float32),
              jax.ShapeDtypeStruct((NC, N_PAD), jnp.float32)),
    mesh=_mesh,
    scratch_types=[
        pltpu.VMEM((NB, B), jnp.int32),
        pltpu.VMEM((NB, B), jnp.int32),
        pltpu.VMEM((128,), jnp.float32),
        pltpu.VMEM((RPT,), jnp.float32),
        pltpu.VMEM_SHARED((N_PAD,), jnp.float32),
        pltpu.VMEM_SHARED((N_PAD,), jnp.float32),
        pltpu.SemaphoreType.DMA,
        pltpu.SemaphoreType.DMA,
    ],
)
def _degree_kernel(src2d_hbm, dst2d_hbm, outs_hbm, outd_hbm,
                   src_v, dst_v, ones_v, zed_v, hsrc_sh, hdst_sh, sa, sb):
    c = lax.axis_index("c")
    s = lax.axis_index("s")
    wid = s * NC + c

    def _init(i, _):
        ones_v[pl.ds(i * 16, 16)] = jnp.full((16,), 1.0, jnp.float32)
        zed_v[pl.ds(i * 16, 16)] = jnp.zeros((16,), jnp.float32)
        return 0
    lax.fori_loop(0, 8, _init, 0)

    def _zed2(i, _):
        zed_v[pl.ds(128 + i * 16, 16)] = jnp.zeros((16,), jnp.float32)
        return 0
    lax.fori_loop(0, (RPT - 128) // 16, _zed2, 0)

    pltpu.sync_copy(zed_v, hsrc_sh.at[pl.ds(s * RPT, RPT)])
    pltpu.sync_copy(zed_v, hdst_sh.at[pl.ds(s * RPT, RPT)])
    plsc.subcore_barrier()

    pltpu.sync_copy(src2d_hbm.at[wid], src_v)
    pltpu.sync_copy(dst2d_hbm.at[wid], dst_v)

    def _ss(j):
        pltpu.async_copy(ones_v.at[pl.ds(0, B)], hsrc_sh.at[src_v.at[j]], sa, add=True)
        pltpu.async_copy(ones_v.at[pl.ds(0, B)], hdst_sh.at[dst_v.at[j]], sb, add=True)

    def _sswait():
        pltpu.make_async_copy(ones_v.at[pl.ds(0, B)], hsrc_sh.at[src_v.at[0]], sa).wait()
        pltpu.make_async_copy(ones_v.at[pl.ds(0, B)], hdst_sh.at[dst_v.at[0]], sb).wait()

    _ss(0)
    _ss(1)
    _ss(2)

    def _step(j, _):
        _ss(j)
        _sswait()
        return 0
    lax.fori_loop(3, NB, _step, 0)
    _sswait()
    _sswait()
    _sswait()

    plsc.subcore_barrier()
    pltpu.sync_copy(hsrc_sh.at[pl.ds(s * RPT, RPT)], outs_hbm.at[c, pl.ds(s * RPT, RPT)])
    pltpu.sync_copy(hdst_sh.at[pl.ds(s * RPT, RPT)], outd_hbm.at[c, pl.ds(s * RPT, RPT)])


@functools.partial(
    pl.kernel,
    out_type=jax.ShapeDtypeStruct((NC, N_PAD, F), jnp.float32),
    mesh=_mesh,
    scratch_types=[
        pltpu.VMEM((2, BM), jnp.int32),
        pltpu.VMEM((2, BM), jnp.int32),
        pltpu.VMEM((2, BM), jnp.int32),
        pltpu.VMEM((2, BM), jnp.int32),
        pltpu.VMEM((BM, F), jnp.float32),
        pltpu.VMEM((BM, F), jnp.float32),
        pltpu.VMEM((16, F), jnp.float32),
        pltpu.VMEM_SHARED((N_PAD, F), jnp.float32),
        pltpu.SemaphoreType.DMA,
        pltpu.SemaphoreType.DMA,
        pltpu.SemaphoreType.DMA,
        pltpu.SemaphoreType.DMA,
        pltpu.SemaphoreType.DMA,
        pltpu.SemaphoreType.DMA,
        pltpu.SemaphoreType.DMA,
        pltpu.SemaphoreType.DMA,
        pltpu.SemaphoreType.DMA,
    ],
)
def _msg_kernel(h_hbm, idx_hbm, out_hbm,
                idx0_v, idx1_v, idx2_v, idx3_v, rows0_v, rows1_v, z_v, acc_sh,
                semi0, semi1, semi2, semi3, semg0, semg1, sems0, sems1, zsem):
    c = lax.axis_index("c")
    s = lax.axis_index("s")
    wid = s * NC + c

    idxb = (idx0_v, idx1_v, idx2_v, idx3_v)
    rowsb = (rows0_v, rows1_v)
    semi = (semi0, semi1, semi2, semi3)
    semg = (semg0, semg1)
    sems = (sems0, sems1)

    def _zinit(i, _):
        for j in range(8):
            z_v[i, pl.ds(j * 16, 16)] = jnp.zeros((16,), jnp.float32)
        return 0
    lax.fori_loop(0, 16, _zinit, 0)

    def _zwait():
        pltpu.make_async_copy(z_v, acc_sh.at[pl.ds(0, 16)], zsem).wait()

    def _ziss(i):
        pltpu.async_copy(z_v, acc_sh.at[pl.ds(s * RPT + i * 16, 16)], zsem)

    _ziss(0)
    _ziss(1)
    _ziss(2)
    _ziss(3)

    def _zacc(i, _):
        _ziss(i)
        _zwait()
        return 0
    lax.fori_loop(4, RPT // 16, _zacc, 0)
    _zwait()
    _zwait()
    _zwait()
    _zwait()
    plsc.subcore_barrier()

    def _iload(j, q):
        pltpu.async_copy(idx_hbm.at[wid, j], idxb[q], semi[q])

    def _iwait(q):
        pltpu.make_async_copy(idx_hbm.at[wid, 0], idxb[q], semi[q]).wait()

    def _gather(q, p):
        pltpu.async_copy(h_hbm.at[idxb[q].at[0]], rowsb[p], semg[p])

    def _gwait(q, p):
        pltpu.make_async_copy(h_hbm.at[idxb[q].at[0]], rowsb[p], semg[p]).wait()

    def _scatter(q, p):
        pltpu.async_copy(rowsb[p], acc_sh.at[idxb[q].at[1]], sems[p], add=True)

    def _swait(q, p):
        pltpu.make_async_copy(rowsb[p], acc_sh.at[idxb[q].at[1]], sems[p]).wait()

    def _stage(j, u, first=False, last=False):
        p = u % 2
        q = u % 4
        if not last:
            _iwait((u + 1) % 4)
        if not first:
            _swait((u + 3) % 4, 1 - p)
        if not last:
            _gather((u + 1) % 4, 1 - p)
        _gwait(q, p)
        _scatter(q, p)
        if not last:
            _iload(jnp.minimum(j + 2, NBM - 1), (u + 2) % 4)

    pltpu.sync_copy(idx_hbm.at[wid, 0], idx0_v)
    _gather(0, 0)
    _iload(1, 1)

    _stage(0, 0, first=True)
    _stage(1, 1)
    _stage(2, 2)
    _stage(3, 3)

    def _quad(jjj, _):
        j0 = 4 * jjj
        _stage(j0, 0)
        _stage(j0 + 1, 1)
        _stage(j0 + 2, 2)
        _stage(j0 + 3, 3)
        return 0
    lax.fori_loop(1, NBM // 4 - 1, _quad, 0)

    _stage(NBM - 4, 0)
    _stage(NBM - 3, 1)
    _stage(NBM - 2, 2)
    _iwait((NBM - 1 + 1) % 4)
    _stage(NBM - 1, 3, last=True)
    _swait(3, 1)

    plsc.subcore_barrier()
    pltpu.sync_copy(acc_sh.at[pl.ds(s * RPT, RPT)],
                    out_hbm.at[c, pl.ds(s * RPT, RPT)])


_R = 2000


def _pre_body(x_ref, d0_ref, d1_ref, w_ref, o_ref):
    deg = d0_ref[...] + d1_ref[...]
    norm = jnp.where(deg > 0, lax.rsqrt(jnp.maximum(deg, 1.0)), 0.0)
    o_ref[...] = jnp.dot(x_ref[...] * norm, w_ref[...],
                         preferred_element_type=jnp.float32)


def _tc_pre(x, dsrc0, dsrc1, W):
    return pl.pallas_call(
        _pre_body,
        grid=(N // _R,),
        in_specs=[
            pl.BlockSpec((_R, F), lambda i: (i, 0)),
            pl.BlockSpec((_R, 1), lambda i: (i, 0)),
            pl.BlockSpec((_R, 1), lambda i: (i, 0)),
            pl.BlockSpec((F, F), lambda i: (0, 0)),
        ],
        out_specs=pl.BlockSpec((_R, F), lambda i: (i, 0)),
        out_shape=jax.ShapeDtypeStruct((N, F), jnp.float32),
    )(x, dsrc0, dsrc1, W)


def _postpre_body(p0_ref, p1_ref, di0_ref, di1_ref, b_ref,
                  do0_ref, do1_ref, w_ref, h_ref, o_ref):
    degi = di0_ref[...] + di1_ref[...]
    normi = jnp.where(degi > 0, lax.rsqrt(jnp.maximum(degi, 1.0)), 0.0)
    p = p0_ref[0].astype(jnp.float32) + p1_ref[0].astype(jnp.float32)
    h = jnp.maximum(p * normi + b_ref[...], 0.0)
    h_ref[...] = h
    dego = do0_ref[...] + do1_ref[...]
    normo = jnp.where(dego > 0, lax.rsqrt(jnp.maximum(dego, 1.0)), 0.0)
    o_ref[...] = jnp.dot(h * normo, w_ref[...],
                         preferred_element_type=jnp.float32)


def _tc_postpre(partials, ddst0, ddst1, b, dsrc0, dsrc1, W):
    return pl.pallas_call(
        _postpre_body,
        grid=(N // _R,),
        in_specs=[
            pl.BlockSpec((1, _R, F), lambda i: (0, i, 0)),
            pl.BlockSpec((1, _R, F), lambda i: (1, i, 0)),
            pl.BlockSpec((_R, 1), lambda i: (i, 0)),
            pl.BlockSpec((_R, 1), lambda i: (i, 0)),
            pl.BlockSpec((1, F), lambda i: (0, 0)),
            pl.BlockSpec((_R, 1), lambda i: (i, 0)),
            pl.BlockSpec((_R, 1), lambda i: (i, 0)),
            pl.BlockSpec((F, F), lambda i: (0, 0)),
        ],
        out_specs=[
            pl.BlockSpec((_R, F), lambda i: (i, 0)),
            pl.BlockSpec((_R, F), lambda i: (i, 0)),
        ],
        out_shape=[
            jax.ShapeDtypeStruct((N, F), jnp.float32),
            jax.ShapeDtypeStruct((N, F), jnp.float32),
        ],
    )(partials, partials, ddst0, ddst1, b, dsrc0, dsrc1, W)


def _post_body(p0_ref, p1_ref, d0_ref, d1_ref, b_ref, o_ref, *, relu):
    deg = d0_ref[...] + d1_ref[...]
    norm = jnp.where(deg > 0, lax.rsqrt(jnp.maximum(deg, 1.0)), 0.0)
    p = p0_ref[0].astype(jnp.float32) + p1_ref[0].astype(jnp.float32)
    h = p * norm + b_ref[...]
    if relu:
        h = jnp.maximum(h, 0.0)
    o_ref[...] = h


def _tc_post(partials, ddst0, ddst1, b, relu):
    return pl.pallas_call(
        functools.partial(_post_body, relu=relu),
        grid=(N // _R,),
        in_specs=[
            pl.BlockSpec((1, _R, F), lambda i: (0, i, 0)),
            pl.BlockSpec((1, _R, F), lambda i: (1, i, 0)),
            pl.BlockSpec((_R, 1), lambda i: (i, 0)),
            pl.BlockSpec((_R, 1), lambda i: (i, 0)),
            pl.BlockSpec((1, F), lambda i: (0, 0)),
        ],
        out_specs=pl.BlockSpec((_R, F), lambda i: (i, 0)),
        out_shape=jax.ShapeDtypeStruct((N, F), jnp.float32),
    )(partials, partials, ddst0, ddst1, b)


def kernel(inputs, edge_index, embedding_layer, W1, b1, W2, b2, W3, b3):
    src2d = edge_index[0].reshape(NW, NB, B)
    dst2d = edge_index[1].reshape(NW, NB, B)
    idx4 = edge_index.reshape(2, NW, NBM, BM).transpose(1, 2, 0, 3)

    dsrc_p, ddst_p = _degree_kernel(src2d, dst2d)
    dsrc0 = dsrc_p[0, :N].reshape(N, 1)
    dsrc1 = dsrc_p[1, :N].reshape(N, 1)
    ddst0 = ddst_p[0, :N].reshape(N, 1)
    ddst1 = ddst_p[1, :N].reshape(N, 1)

    b1r = b1.reshape(1, F)
    b2r = b2.reshape(1, F)
    b3r = b3.reshape(1, F)

    pre1 = _tc_pre(inputs, dsrc0, dsrc1, W1)
    p1_ = _msg_kernel(pre1, idx4)
    h1, pre2 = _tc_postpre(p1_, ddst0, ddst1, b1r, dsrc0, dsrc1, W2)

    p2_ = _msg_kernel(pre2, idx4)
    h2, pre3 = _tc_postpre(p2_, ddst0, ddst1, b2r, dsrc0, dsrc1, W3)

    p3_ = _msg_kernel(pre3, idx4)
    h3 = _tc_post(p3_, ddst0, ddst1, b3r, relu=False)

    emb = jnp.where(embedding_layer == 1, h1,
                    jnp.where(embedding_layer == 2, h2, h3))
    return (h3, emb, inputs)
